# Initial kernel scaffold; baseline (speedup 1.0000x reference)
import jax
import jax.numpy as jnp
from jax.experimental import pallas as pl


def kernel(node_features, edges, W1, b1, W2, b2, W3, b3):
    raise NotImplementedError("write your pallas kernel here")


# speedup vs baseline: 81.5484x; 81.5484x over previous
import functools

import jax
import jax.numpy as jnp
from jax.experimental import pallas as pl


def _dinv_body(e_ref, out_ref, *, ns):
    s = pl.program_id(0)

    @pl.when(s == 0)
    def _():
        out_ref[...] = jnp.zeros_like(out_ref)

    out_ref[...] += jnp.sum(e_ref[...], axis=0, keepdims=True)

    @pl.when(s == ns - 1)
    def _():
        out_ref[...] = jax.lax.rsqrt(out_ref[...] + 1.0)


def _colsum_dinv(e, bs):
    n = e.shape[0]
    ns = n // bs
    return pl.pallas_call(
        functools.partial(_dinv_body, ns=ns),
        grid=(ns,),
        in_specs=[pl.BlockSpec((bs, n), lambda s: (s, 0))],
        out_specs=pl.BlockSpec((1, n), lambda s: (0, 0)),
        out_shape=jax.ShapeDtypeStruct((1, n), jnp.float32),
    )(e)


def _proj_body(wt_ref, x_ref, dinv_ref, out_ref):
    out_ref[...] = (
        jnp.dot(wt_ref[...], x_ref[...], preferred_element_type=jnp.float32)
        * dinv_ref[...]
    )


def _proj(wt, x_t, dinv_p, bp):
    h, np_ = wt.shape[0], x_t.shape[1]
    return pl.pallas_call(
        _proj_body,
        grid=(np_ // bp,),
        in_specs=[
            pl.BlockSpec((h, wt.shape[1]), lambda i: (0, 0)),
            pl.BlockSpec((wt.shape[1], bp), lambda i: (0, i)),
            pl.BlockSpec((1, bp), lambda i: (0, i)),
        ],
        out_specs=pl.BlockSpec((h, bp), lambda i: (0, i)),
        out_shape=jax.ShapeDtypeStruct((h, np_), jnp.float32),
    )(wt, x_t, dinv_p)


def _agg_body(
    y_ref, e_ref, yself_ref, dinv_ref, b_ref, out_ref, *, n, ns, bs, bd, relu_out
):
    d, s = pl.program_id(0), pl.program_id(1)

    @pl.when(s == 0)
    def _():
        out_ref[...] = jnp.zeros_like(out_ref)

    row_lim = n - s * bs

    @pl.when(row_lim >= bs)
    def _():
        out_ref[...] += jnp.dot(
            y_ref[...], e_ref[...], preferred_element_type=jnp.float32
        )

    @pl.when(row_lim < bs)
    def _():
        rows = jax.lax.broadcasted_iota(jnp.int32, e_ref.shape, 0)
        e = jnp.where(rows < row_lim, e_ref[...], 0.0)
        out_ref[...] += jnp.dot(y_ref[...], e, preferred_element_type=jnp.float32)

    @pl.when(s == ns - 1)
    def _():
        r = dinv_ref[...] * (out_ref[...] + yself_ref[...]) + b_ref[...]
        if relu_out:
            r = jnp.maximum(r, 0.0)
        lanes = jax.lax.broadcasted_iota(jnp.int32, r.shape, 1)
        out_ref[...] = jnp.where(lanes < n - d * bd, r, 0.0)


def _agg(y_t, e, dinv_p, b_col, relu_out, bs, bd):
    h, np_ = y_t.shape
    n = e.shape[0]
    nd, ns = np_ // bd, np_ // bs
    return pl.pallas_call(
        functools.partial(
            _agg_body, n=n, ns=ns, bs=bs, bd=bd, relu_out=relu_out
        ),
        grid=(nd, ns),
        in_specs=[
            pl.BlockSpec((h, bs), lambda d, s: (0, s)),
            pl.BlockSpec((bs, bd), lambda d, s: (s, d)),
            pl.BlockSpec((h, bd), lambda d, s: (0, d)),
            pl.BlockSpec((1, bd), lambda d, s: (0, d)),
            pl.BlockSpec((h, 1), lambda d, s: (0, 0)),
        ],
        out_specs=pl.BlockSpec((h, bd), lambda d, s: (0, d)),
        out_shape=jax.ShapeDtypeStruct((h, np_), jnp.float32),
    )(y_t, e, y_t, dinv_p, b_col)


def kernel(node_features, edges, W1, b1, W2, b2, W3, b3):
    n, dim = node_features.shape
    bs = min(2048, -(-n // 128) * 128)
    np_ = -(-n // bs) * bs
    bd = min(1024, np_)

    csum_bs = 1
    for c in range(8, min(n, 400) + 1, 8):
        if n % c == 0:
            csum_bs = c
    dinv = _colsum_dinv(edges, bs=csum_bs)

    pad = np_ - n
    dinv_p = jnp.pad(dinv, ((0, 0), (0, pad)))
    x_t = jnp.pad(node_features.T, ((0, 0), (0, pad)))

    out = x_t
    layers = [(W1, b1), (W2, b2), (W3, b3)]
    for i, (w, b) in enumerate(layers):
        y = _proj(w.T, out, dinv_p, bp=bs)
        out = _agg(
            y, edges, dinv_p, b.reshape(-1, 1),
            relu_out=(i != len(layers) - 1), bs=bs, bd=bd,
        )
    return out[:, :n].T


# speedup vs baseline: 92.4095x; 1.1332x over previous
import functools

import jax
import jax.numpy as jnp
from jax.experimental import pallas as pl


_ETYPE = jnp.bfloat16


def _dinv_body(e_ref, out_ref, ec_ref, *, ns):
    s = pl.program_id(0)

    @pl.when(s == 0)
    def _():
        out_ref[...] = jnp.zeros_like(out_ref)

    e = e_ref[...]
    out_ref[...] += jnp.sum(e, axis=0, keepdims=True)
    ec_ref[...] = e.astype(_ETYPE)

    @pl.when(s == ns - 1)
    def _():
        out_ref[...] = jax.lax.rsqrt(out_ref[...] + 1.0)


def _colsum_dinv(e, bs):
    n = e.shape[0]
    ns = n // bs
    return pl.pallas_call(
        functools.partial(_dinv_body, ns=ns),
        grid=(ns,),
        in_specs=[pl.BlockSpec((bs, n), lambda s: (s, 0))],
        out_specs=[
            pl.BlockSpec((1, n), lambda s: (0, 0)),
            pl.BlockSpec((bs, n), lambda s: (s, 0)),
        ],
        out_shape=[
            jax.ShapeDtypeStruct((1, n), jnp.float32),
            jax.ShapeDtypeStruct((n, n), _ETYPE),
        ],
    )(e)


def _proj_body(wt_ref, x_ref, dinv_ref, out_ref):
    out_ref[...] = (
        jnp.dot(wt_ref[...], x_ref[...], preferred_element_type=jnp.float32)
        * dinv_ref[...]
    )


def _proj(wt, x_t, dinv_p, bp):
    h, np_ = wt.shape[0], x_t.shape[1]
    return pl.pallas_call(
        _proj_body,
        grid=(np_ // bp,),
        in_specs=[
            pl.BlockSpec((h, wt.shape[1]), lambda i: (0, 0)),
            pl.BlockSpec((wt.shape[1], bp), lambda i: (0, i)),
            pl.BlockSpec((1, bp), lambda i: (0, i)),
        ],
        out_specs=pl.BlockSpec((h, bp), lambda i: (0, i)),
        out_shape=jax.ShapeDtypeStruct((h, np_), jnp.float32),
    )(wt, x_t, dinv_p)


def _agg_body(
    y_ref, e_ref, yself_ref, dinv_ref, b_ref, out_ref, *, n, ns, bs, bd, relu_out
):
    d, s = pl.program_id(0), pl.program_id(1)

    @pl.when(s == 0)
    def _():
        out_ref[...] = jnp.zeros_like(out_ref)

    row_lim = n - s * bs
    y = y_ref[...].astype(e_ref.dtype)

    @pl.when(row_lim >= bs)
    def _():
        out_ref[...] += jnp.dot(
            y, e_ref[...], preferred_element_type=jnp.float32
        )

    @pl.when(row_lim < bs)
    def _():
        rows = jax.lax.broadcasted_iota(jnp.int32, e_ref.shape, 0)
        e = jnp.where(rows < row_lim, e_ref[...], jnp.zeros((), e_ref.dtype))
        out_ref[...] += jnp.dot(y, e, preferred_element_type=jnp.float32)

    @pl.when(s == ns - 1)
    def _():
        r = dinv_ref[...] * (out_ref[...] + yself_ref[...]) + b_ref[...]
        if relu_out:
            r = jnp.maximum(r, 0.0)
        lanes = jax.lax.broadcasted_iota(jnp.int32, r.shape, 1)
        out_ref[...] = jnp.where(lanes < n - d * bd, r, 0.0)


def _agg(y_t, e, dinv_p, b_col, relu_out, bs, bd):
    h, np_ = y_t.shape
    n = e.shape[0]
    nd, ns = np_ // bd, np_ // bs
    return pl.pallas_call(
        functools.partial(
            _agg_body, n=n, ns=ns, bs=bs, bd=bd, relu_out=relu_out
        ),
        grid=(nd, ns),
        in_specs=[
            pl.BlockSpec((h, bs), lambda d, s: (0, s)),
            pl.BlockSpec((bs, bd), lambda d, s: (s, d)),
            pl.BlockSpec((h, bd), lambda d, s: (0, d)),
            pl.BlockSpec((1, bd), lambda d, s: (0, d)),
            pl.BlockSpec((h, 1), lambda d, s: (0, 0)),
        ],
        out_specs=pl.BlockSpec((h, bd), lambda d, s: (0, d)),
        out_shape=jax.ShapeDtypeStruct((h, np_), jnp.float32),
    )(y_t, e, y_t, dinv_p, b_col)


def kernel(node_features, edges, W1, b1, W2, b2, W3, b3):
    n, dim = node_features.shape
    bs = min(2048, -(-n // 128) * 128)
    np_ = -(-n // bs) * bs
    bd = min(1024, np_)

    csum_bs = 1
    for c in range(16, min(n, 400) + 1, 16):
        if n % c == 0:
            csum_bs = c
    dinv, e_c = _colsum_dinv(edges, bs=csum_bs)

    pad = np_ - n
    dinv_p = jnp.pad(dinv, ((0, 0), (0, pad)))
    x_t = jnp.pad(node_features.T, ((0, 0), (0, pad)))

    out = x_t
    layers = [(W1, b1), (W2, b2), (W3, b3)]
    for i, (w, b) in enumerate(layers):
        y = _proj(w.T, out, dinv_p, bp=bs)
        out = _agg(
            y, e_c, dinv_p, b.reshape(-1, 1),
            relu_out=(i != len(layers) - 1), bs=bs, bd=bd,
        )
    return out[:, :n].T


# speedup vs baseline: 106.8712x; 1.1565x over previous
import functools

import jax
import jax.numpy as jnp
from jax.experimental import pallas as pl


_ETYPE = jnp.bfloat16


def _dinv_body(e_ref, out_ref, ec_ref, *, ns):
    s = pl.program_id(0)

    @pl.when(s == 0)
    def _():
        out_ref[...] = jnp.zeros_like(out_ref)

    e = e_ref[...]
    out_ref[...] += jnp.sum(e, axis=0, keepdims=True)
    ec_ref[...] = e.astype(_ETYPE)

    @pl.when(s == ns - 1)
    def _():
        out_ref[...] = jax.lax.rsqrt(out_ref[...] + 1.0)


def _colsum_dinv(e, bs):
    n = e.shape[0]
    ns = n // bs
    return pl.pallas_call(
        functools.partial(_dinv_body, ns=ns),
        grid=(ns,),
        in_specs=[pl.BlockSpec((bs, n), lambda s: (s, 0))],
        out_specs=[
            pl.BlockSpec((1, n), lambda s: (0, 0)),
            pl.BlockSpec((bs, n), lambda s: (s, 0)),
        ],
        out_shape=[
            jax.ShapeDtypeStruct((1, n), jnp.float32),
            jax.ShapeDtypeStruct((n, n), _ETYPE),
        ],
    )(e)


def _proj_body(wt_ref, x_ref, dinv_ref, out_ref):
    out_ref[...] = (
        jnp.dot(wt_ref[...], x_ref[...], preferred_element_type=jnp.float32)
        * dinv_ref[...]
    ).astype(out_ref.dtype)


def _proj(wt, x_t, dinv_p, bp):
    h, np_ = wt.shape[0], x_t.shape[1]
    return pl.pallas_call(
        _proj_body,
        grid=(np_ // bp,),
        in_specs=[
            pl.BlockSpec((h, wt.shape[1]), lambda i: (0, 0)),
            pl.BlockSpec((wt.shape[1], bp), lambda i: (0, i)),
            pl.BlockSpec((1, bp), lambda i: (0, i)),
        ],
        out_specs=pl.BlockSpec((h, bp), lambda i: (0, i)),
        out_shape=jax.ShapeDtypeStruct((h, np_), _ETYPE),
    )(wt, x_t, dinv_p)


def _agg_body(
    y_ref, e_ref, yself_ref, dinv_ref, b_ref, out_ref, *, n, ns, bs, bd, relu_out
):
    d, s = pl.program_id(0), pl.program_id(1)

    @pl.when(s == 0)
    def _():
        out_ref[...] = jnp.zeros_like(out_ref)

    row_lim = n - s * bs

    @pl.when(row_lim >= bs)
    def _():
        out_ref[...] += jnp.dot(
            y_ref[...], e_ref[...], preferred_element_type=jnp.float32
        )

    @pl.when(row_lim < bs)
    def _():
        rows = jax.lax.broadcasted_iota(jnp.int32, e_ref.shape, 0)
        e = jnp.where(rows < row_lim, e_ref[...], jnp.zeros((), e_ref.dtype))
        out_ref[...] += jnp.dot(
            y_ref[...], e, preferred_element_type=jnp.float32
        )

    @pl.when(s == ns - 1)
    def _():
        yself = yself_ref[...].astype(jnp.float32)
        r = dinv_ref[...] * (out_ref[...] + yself) + b_ref[...]
        if relu_out:
            r = jnp.maximum(r, 0.0)
        lanes = jax.lax.broadcasted_iota(jnp.int32, r.shape, 1)
        out_ref[...] = jnp.where(lanes < n - d * bd, r, 0.0)


def _agg(y_t, e, dinv_p, b_col, relu_out, bs, bd):
    h, np_ = y_t.shape
    n = e.shape[0]
    nd, ns = np_ // bd, np_ // bs
    return pl.pallas_call(
        functools.partial(
            _agg_body, n=n, ns=ns, bs=bs, bd=bd, relu_out=relu_out
        ),
        grid=(nd, ns),
        in_specs=[
            pl.BlockSpec((h, bs), lambda d, s: (0, s)),
            pl.BlockSpec((bs, bd), lambda d, s: (s, d)),
            pl.BlockSpec((h, bd), lambda d, s: (0, d)),
            pl.BlockSpec((1, bd), lambda d, s: (0, d)),
            pl.BlockSpec((h, 1), lambda d, s: (0, 0)),
        ],
        out_specs=pl.BlockSpec((h, bd), lambda d, s: (0, d)),
        out_shape=jax.ShapeDtypeStruct((h, np_), jnp.float32),
    )(y_t, e, y_t, dinv_p, b_col)


def kernel(node_features, edges, W1, b1, W2, b2, W3, b3):
    n, dim = node_features.shape
    bs = min(2048, -(-n // 128) * 128)
    np_ = -(-n // bs) * bs
    bd = min(2048, np_)

    csum_bs = 1
    for c in range(16, min(n, 400) + 1, 16):
        if n % c == 0:
            csum_bs = c
    dinv, e_c = _colsum_dinv(edges, bs=csum_bs)

    pad = np_ - n
    dinv_p = jnp.pad(dinv, ((0, 0), (0, pad)))
    x_t = jnp.pad(node_features.T, ((0, 0), (0, pad)))

    out = x_t
    layers = [(W1, b1), (W2, b2), (W3, b3)]
    for i, (w, b) in enumerate(layers):
        y = _proj(w.T, out, dinv_p, bp=bs)
        out = _agg(
            y, e_c, dinv_p, b.reshape(-1, 1),
            relu_out=(i != len(layers) - 1), bs=bs, bd=bd,
        )
    return out[:, :n].T


# speedup vs baseline: 110.7402x; 1.0362x over previous
import functools

import jax
import jax.numpy as jnp
from jax.experimental import pallas as pl
from jax.experimental.pallas import tpu as pltpu

_ETYPE = jnp.bfloat16


def _dinv_body(e_ref, out_ref, ec_ref, *, ns):
    s = pl.program_id(0)

    @pl.when(s == 0)
    def _():
        out_ref[...] = jnp.zeros_like(out_ref)

    e = e_ref[...]
    out_ref[...] += jnp.sum(e, axis=0, keepdims=True)
    ec_ref[...] = e.astype(_ETYPE)

    @pl.when(s == ns - 1)
    def _():
        out_ref[...] = jax.lax.rsqrt(out_ref[...] + 1.0)


def _colsum_dinv(e, bs):
    n = e.shape[0]
    ns = n // bs
    return pl.pallas_call(
        functools.partial(_dinv_body, ns=ns),
        grid=(ns,),
        in_specs=[pl.BlockSpec((bs, n), lambda s: (s, 0))],
        out_specs=[
            pl.BlockSpec((1, n), lambda s: (0, 0)),
            pl.BlockSpec((bs, n), lambda s: (s, 0)),
        ],
        out_shape=[
            jax.ShapeDtypeStruct((1, n), jnp.float32),
            jax.ShapeDtypeStruct((n, n), _ETYPE),
        ],
    )(e)


def _layer_body(
    wt_ref, x_ref, e_ref, dinv_s_ref, dinv_d_ref, b_ref, out_ref, ybuf_ref,
    *, n, ns, bs, bd, relu_out,
):
    d, s = pl.program_id(0), pl.program_id(1)

    @pl.when(s == 0)
    def _():
        out_ref[...] = jnp.zeros_like(out_ref)

    y = (
        jnp.dot(wt_ref[...], x_ref[...], preferred_element_type=jnp.float32)
        * dinv_s_ref[...]
    ).astype(_ETYPE)

    @pl.when(s == d)
    def _():
        ybuf_ref[...] = y

    row_lim = n - s * bs

    @pl.when(row_lim >= bs)
    def _():
        out_ref[...] += jnp.dot(
            y, e_ref[...], preferred_element_type=jnp.float32
        )

    @pl.when(row_lim < bs)
    def _():
        rows = jax.lax.broadcasted_iota(jnp.int32, e_ref.shape, 0)
        e = jnp.where(rows < row_lim, e_ref[...], jnp.zeros((), e_ref.dtype))
        out_ref[...] += jnp.dot(y, e, preferred_element_type=jnp.float32)

    @pl.when(s == ns - 1)
    def _():
        yself = ybuf_ref[...].astype(jnp.float32)
        r = dinv_d_ref[...] * (out_ref[...] + yself) + b_ref[...]
        if relu_out:
            r = jnp.maximum(r, 0.0)
        lanes = jax.lax.broadcasted_iota(jnp.int32, r.shape, 1)
        out_ref[...] = jnp.where(lanes < n - d * bd, r, 0.0)


def _layer(wt, x_t, e_c, dinv_p, b_col, relu_out, bs, bd):
    h, np_ = x_t.shape
    n = e_c.shape[0]
    nd, ns = np_ // bd, np_ // bs
    return pl.pallas_call(
        functools.partial(
            _layer_body, n=n, ns=ns, bs=bs, bd=bd, relu_out=relu_out
        ),
        grid=(nd, ns),
        in_specs=[
            pl.BlockSpec((h, h), lambda d, s: (0, 0)),
            pl.BlockSpec((h, bs), lambda d, s: (0, s)),
            pl.BlockSpec((bs, bd), lambda d, s: (s, d)),
            pl.BlockSpec((1, bs), lambda d, s: (0, s)),
            pl.BlockSpec((1, bd), lambda d, s: (0, d)),
            pl.BlockSpec((h, 1), lambda d, s: (0, 0)),
        ],
        out_specs=pl.BlockSpec((h, bd), lambda d, s: (0, d)),
        out_shape=jax.ShapeDtypeStruct((h, np_), jnp.float32),
        scratch_shapes=[pltpu.VMEM((h, bd), _ETYPE)],
    )(wt, x_t, e_c, dinv_p, dinv_p, b_col)


def kernel(node_features, edges, W1, b1, W2, b2, W3, b3):
    n, dim = node_features.shape
    bs = min(2560, -(-n // 128) * 128)
    np_ = -(-n // bs) * bs
    bd = bs

    csum_bs = 1
    for c in range(16, min(n, 400) + 1, 16):
        if n % c == 0:
            csum_bs = c
    dinv, e_c = _colsum_dinv(edges, bs=csum_bs)

    pad = np_ - n
    dinv_p = jnp.pad(dinv, ((0, 0), (0, pad)))
    x_t = jnp.pad(node_features.T, ((0, 0), (0, pad)))

    out = x_t
    layers = [(W1, b1), (W2, b2), (W3, b3)]
    for i, (w, b) in enumerate(layers):
        out = _layer(
            w.T, out, e_c, dinv_p, b.reshape(-1, 1),
            relu_out=(i != len(layers) - 1), bs=bs, bd=bd,
        )
    return out[:, :n].T


# speedup vs baseline: 114.8381x; 1.0370x over previous
import functools

import jax
import jax.numpy as jnp
from jax.experimental import pallas as pl
from jax.experimental.pallas import tpu as pltpu

_ETYPE = jnp.bfloat16


def _dinv_body(e_ref, out_ref, ec_ref, *, ns):
    s = pl.program_id(0)

    @pl.when(s == 0)
    def _():
        out_ref[...] = jnp.zeros_like(out_ref)

    e = e_ref[...]
    out_ref[...] += jnp.sum(e, axis=0, keepdims=True)
    ec_ref[...] = e.astype(_ETYPE)

    @pl.when(s == ns - 1)
    def _():
        out_ref[...] = jax.lax.rsqrt(out_ref[...] + 1.0)


def _colsum_dinv(e, bs):
    n = e.shape[0]
    ns = n // bs
    return pl.pallas_call(
        functools.partial(_dinv_body, ns=ns),
        grid=(ns,),
        in_specs=[pl.BlockSpec((bs, n), lambda s: (s, 0))],
        out_specs=[
            pl.BlockSpec((1, n), lambda s: (0, 0)),
            pl.BlockSpec((bs, n), lambda s: (s, 0)),
        ],
        out_shape=[
            jax.ShapeDtypeStruct((1, n), jnp.float32),
            jax.ShapeDtypeStruct((n, n), _ETYPE),
        ],
    )(e)


def _layer_body(
    wt_ref, x_ref, e_ref, dinv_s_ref, dinv_d_ref, b_ref, out_ref,
    acc_ref, ybuf_ref,
    *, n, ns, bs, bd, relu_out, in_natural, out_natural,
):
    d, s = pl.program_id(0), pl.program_id(1)
    row_lim = n - s * bs

    wt = wt_ref[...].astype(_ETYPE)
    if in_natural:
        rows = jax.lax.broadcasted_iota(jnp.int32, x_ref.shape, 0)
        x = jnp.where(rows < row_lim, x_ref[...], 0.0).astype(_ETYPE)
        p = jax.lax.dot_general(
            wt, x, (((0,), (1,)), ((), ())), preferred_element_type=jnp.float32
        )
    else:
        p = jnp.dot(
            wt, x_ref[...].astype(_ETYPE), preferred_element_type=jnp.float32
        )
    y = (p * dinv_s_ref[...]).astype(_ETYPE)

    @pl.when(s == 0)
    def _():
        acc_ref[...] = jnp.zeros_like(acc_ref)

    @pl.when(s == d)
    def _():
        ybuf_ref[...] = y

    @pl.when(row_lim >= bs)
    def _():
        acc_ref[...] += jnp.dot(y, e_ref[...], preferred_element_type=jnp.float32)

    @pl.when(row_lim < bs)
    def _():
        erows = jax.lax.broadcasted_iota(jnp.int32, e_ref.shape, 0)
        e = jnp.where(erows < row_lim, e_ref[...], jnp.zeros((), e_ref.dtype))
        acc_ref[...] += jnp.dot(y, e, preferred_element_type=jnp.float32)

    @pl.when(s == ns - 1)
    def _():
        yself = ybuf_ref[...].astype(jnp.float32)
        r = dinv_d_ref[...] * (acc_ref[...] + yself) + b_ref[...]
        if relu_out:
            r = jnp.maximum(r, 0.0)
        if out_natural:
            out_ref[...] = r.T
        else:
            lanes = jax.lax.broadcasted_iota(jnp.int32, r.shape, 1)
            out_ref[...] = jnp.where(lanes < n - d * bd, r, 0.0)


def _layer(wmat, x, e_c, dinv_p, b_col, relu_out, in_natural, out_natural, bs, bd):
    d_in, h = wmat.shape if in_natural else wmat.shape[::-1]
    n = e_c.shape[0]
    np_ = dinv_p.shape[1]
    nd, ns = np_ // bd, np_ // bs
    if out_natural:
        out_spec = pl.BlockSpec((bd, h), lambda d, s: (d, 0))
        out_shape = jax.ShapeDtypeStruct((n, h), jnp.float32)
    else:
        out_spec = pl.BlockSpec((h, bd), lambda d, s: (0, d))
        out_shape = jax.ShapeDtypeStruct((h, np_), jnp.float32)
    if in_natural:
        x_spec = pl.BlockSpec((bs, d_in), lambda d, s: (s, 0))
    else:
        x_spec = pl.BlockSpec((d_in, bs), lambda d, s: (0, s))
    return pl.pallas_call(
        functools.partial(
            _layer_body, n=n, ns=ns, bs=bs, bd=bd, relu_out=relu_out,
            in_natural=in_natural, out_natural=out_natural,
        ),
        grid=(nd, ns),
        in_specs=[
            pl.BlockSpec(wmat.shape, lambda d, s: (0, 0)),
            x_spec,
            pl.BlockSpec((bs, bd), lambda d, s: (s, d)),
            pl.BlockSpec((1, bs), lambda d, s: (0, s)),
            pl.BlockSpec((1, bd), lambda d, s: (0, d)),
            pl.BlockSpec((h, 1), lambda d, s: (0, 0)),
        ],
        out_specs=out_spec,
        out_shape=out_shape,
        scratch_shapes=[
            pltpu.VMEM((h, bd), jnp.float32),
            pltpu.VMEM((h, bd), _ETYPE),
        ],
    )(wmat, x, e_c, dinv_p, dinv_p, b_col)


def kernel(node_features, edges, W1, b1, W2, b2, W3, b3):
    n, dim = node_features.shape
    bs = min(2560, -(-n // 128) * 128)
    np_ = -(-n // bs) * bs
    bd = bs

    csum_bs = 1
    for c in range(16, min(n, 400) + 1, 16):
        if n % c == 0:
            csum_bs = c
    dinv, e_c = _colsum_dinv(edges, bs=csum_bs)

    dinv_p = jnp.pad(dinv, ((0, 0), (0, np_ - n)))

    out = node_features
    layers = [(W1, b1), (W2, b2), (W3, b3)]
    for i, (w, b) in enumerate(layers):
        last = i == len(layers) - 1
        out = _layer(
            w if i == 0 else w.T, out, e_c, dinv_p, b.reshape(-1, 1),
            relu_out=not last, in_natural=(i == 0), out_natural=last,
            bs=bs, bd=bd,
        )
    return out


# speedup vs baseline: 117.3264x; 1.0217x over previous
import functools

import jax
import jax.numpy as jnp
from jax.experimental import pallas as pl
from jax.experimental.pallas import tpu as pltpu

_ETYPE = jnp.bfloat16


def _dinv_body(e_ref, out_ref, ec_ref, *, ns):
    s = pl.program_id(0)

    @pl.when(s == 0)
    def _():
        out_ref[...] = jnp.zeros_like(out_ref)

    e = e_ref[...]
    out_ref[...] += jnp.sum(e, axis=0, keepdims=True)
    ec_ref[...] = e.astype(_ETYPE)

    @pl.when(s == ns - 1)
    def _():
        out_ref[...] = jax.lax.rsqrt(out_ref[...] + 1.0)


def _colsum_dinv(e, bs):
    n = e.shape[0]
    ns = n // bs
    return pl.pallas_call(
        functools.partial(_dinv_body, ns=ns),
        grid=(ns,),
        in_specs=[pl.BlockSpec((bs, n), lambda s: (s, 0))],
        out_specs=[
            pl.BlockSpec((1, n), lambda s: (0, 0)),
            pl.BlockSpec((bs, n), lambda s: (s, 0)),
        ],
        out_shape=[
            jax.ShapeDtypeStruct((1, n), jnp.float32),
            jax.ShapeDtypeStruct((n, n), _ETYPE),
        ],
    )(e)


def _layer_body(
    wt_ref, x_ref, e_ref, dinv_s_ref, dinv_d_ref, b_ref, out_ref,
    acc_ref, ybuf_ref,
    *, n, ns, bs, bd, relu_out, in_natural, out_natural,
):
    d, s = pl.program_id(0), pl.program_id(1)
    row_lim = n - s * bs

    wt = wt_ref[...].astype(_ETYPE)
    if in_natural:
        rows = jax.lax.broadcasted_iota(jnp.int32, x_ref.shape, 0)
        x = jnp.where(rows < row_lim, x_ref[...], 0.0).astype(_ETYPE)
        p = jax.lax.dot_general(
            wt, x, (((0,), (1,)), ((), ())), preferred_element_type=jnp.float32
        )
    else:
        p = jnp.dot(
            wt, x_ref[...].astype(_ETYPE), preferred_element_type=jnp.float32
        )
    y = (p * dinv_s_ref[...]).astype(_ETYPE)

    @pl.when(s == 0)
    def _():
        acc_ref[...] = jnp.zeros_like(acc_ref)

    r_ds = bd // bs

    @pl.when((s >= d * r_ds) & (s < (d + 1) * r_ds))
    def _():
        ybuf_ref[:, pl.ds((s - d * r_ds) * bs, bs)] = y

    @pl.when(row_lim >= bs)
    def _():
        acc_ref[...] += jnp.dot(y, e_ref[...], preferred_element_type=jnp.float32)

    @pl.when(row_lim < bs)
    def _():
        erows = jax.lax.broadcasted_iota(jnp.int32, e_ref.shape, 0)
        e = jnp.where(erows < row_lim, e_ref[...], jnp.zeros((), e_ref.dtype))
        acc_ref[...] += jnp.dot(y, e, preferred_element_type=jnp.float32)

    @pl.when(s == ns - 1)
    def _():
        yself = ybuf_ref[...].astype(jnp.float32)
        r = dinv_d_ref[...] * (acc_ref[...] + yself) + b_ref[...]
        if relu_out:
            r = jnp.maximum(r, 0.0)
        if out_natural:
            out_ref[...] = r.T
        else:
            lanes = jax.lax.broadcasted_iota(jnp.int32, r.shape, 1)
            out_ref[...] = jnp.where(lanes < n - d * bd, r, 0.0)


def _layer(wmat, x, e_c, dinv_p, b_col, relu_out, in_natural, out_natural, bs, bd):
    d_in, h = wmat.shape if in_natural else wmat.shape[::-1]
    n = e_c.shape[0]
    np_ = dinv_p.shape[1]
    nd, ns = np_ // bd, np_ // bs
    if out_natural:
        out_spec = pl.BlockSpec((bd, h), lambda d, s: (d, 0))
        out_shape = jax.ShapeDtypeStruct((n, h), jnp.float32)
    else:
        out_spec = pl.BlockSpec((h, bd), lambda d, s: (0, d))
        out_shape = jax.ShapeDtypeStruct((h, np_), jnp.float32)
    if in_natural:
        x_spec = pl.BlockSpec((bs, d_in), lambda d, s: (s, 0))
    else:
        x_spec = pl.BlockSpec((d_in, bs), lambda d, s: (0, s))
    return pl.pallas_call(
        functools.partial(
            _layer_body, n=n, ns=ns, bs=bs, bd=bd, relu_out=relu_out,
            in_natural=in_natural, out_natural=out_natural,
        ),
        grid=(nd, ns),
        in_specs=[
            pl.BlockSpec(wmat.shape, lambda d, s: (0, 0)),
            x_spec,
            pl.BlockSpec((bs, bd), lambda d, s: (s, d)),
            pl.BlockSpec((1, bs), lambda d, s: (0, s)),
            pl.BlockSpec((1, bd), lambda d, s: (0, d)),
            pl.BlockSpec((h, 1), lambda d, s: (0, 0)),
        ],
        out_specs=out_spec,
        out_shape=out_shape,
        scratch_shapes=[
            pltpu.VMEM((h, bd), jnp.float32),
            pltpu.VMEM((h, bd), _ETYPE),
        ],
    )(wmat, x, e_c, dinv_p, dinv_p, b_col)


def kernel(node_features, edges, W1, b1, W2, b2, W3, b3):
    n, dim = node_features.shape
    bs = min(1280, -(-n // 128) * 128)
    np_ = -(-n // bs) * bs
    bd = min(4 * bs, np_)

    csum_bs = 1
    for c in range(16, min(n, 400) + 1, 16):
        if n % c == 0:
            csum_bs = c
    dinv, e_c = _colsum_dinv(edges, bs=csum_bs)

    dinv_p = jnp.pad(dinv, ((0, 0), (0, np_ - n)))

    out = node_features
    layers = [(W1, b1), (W2, b2), (W3, b3)]
    for i, (w, b) in enumerate(layers):
        last = i == len(layers) - 1
        out = _layer(
            w if i == 0 else w.T, out, e_c, dinv_p, b.reshape(-1, 1),
            relu_out=not last, in_natural=(i == 0), out_natural=last,
            bs=bs, bd=bd,
        )
    return out


# speedup vs baseline: 118.5092x; 1.0101x over previous
import functools

import jax
import jax.numpy as jnp
from jax.experimental import pallas as pl
from jax.experimental.pallas import tpu as pltpu

_ETYPE = jnp.bfloat16


def _dinv_body(e_ref, out_ref, ec_ref, *, ns):
    s = pl.program_id(0)

    @pl.when(s == 0)
    def _():
        out_ref[...] = jnp.zeros_like(out_ref)

    e = e_ref[...]
    out_ref[...] += jnp.sum(e, axis=0, keepdims=True)
    ec_ref[...] = e.astype(_ETYPE)

    @pl.when(s == ns - 1)
    def _():
        out_ref[...] = jax.lax.rsqrt(out_ref[...] + 1.0)


def _colsum_dinv(e, bs):
    n = e.shape[0]
    ns = n // bs
    return pl.pallas_call(
        functools.partial(_dinv_body, ns=ns),
        grid=(ns,),
        in_specs=[pl.BlockSpec((bs, n), lambda s: (s, 0))],
        out_specs=[
            pl.BlockSpec((1, n), lambda s: (0, 0)),
            pl.BlockSpec((bs, n), lambda s: (s, 0)),
        ],
        out_shape=[
            jax.ShapeDtypeStruct((1, n), jnp.float32),
            jax.ShapeDtypeStruct((n, n), _ETYPE),
        ],
    )(e)


def _gcn_body(
    wt_ref, xin_ref, e_ref, dinv_s_ref, dinv_d_ref, b_ref, out_ref,
    xa_ref, xb_ref, acc_ref, ybuf_ref,
    *, n, nl, nd, ns, bs, bd,
):
    l, d, s = pl.program_id(0), pl.program_id(1), pl.program_id(2)
    row_lim = n - s * bs

    wt = wt_ref[0].astype(_ETYPE)

    def proj_layer1():
        rows = jax.lax.broadcasted_iota(jnp.int32, xin_ref.shape, 0)
        x = jnp.where(rows < row_lim, xin_ref[...], 0.0).astype(_ETYPE)
        return jax.lax.dot_general(
            wt, x, (((1,), (1,)), ((), ())), preferred_element_type=jnp.float32
        )

    def proj_resident(buf_ref):
        def f():
            x = buf_ref[:, pl.ds(s * bs, bs)].astype(_ETYPE)
            return jnp.dot(wt, x, preferred_element_type=jnp.float32)

        return f

    p = jax.lax.cond(
        l == 0,
        proj_layer1,
        lambda: jax.lax.cond(
            l == 1, proj_resident(xa_ref), proj_resident(xb_ref)
        ),
    )
    y = (p * dinv_s_ref[...]).astype(_ETYPE)

    @pl.when(s == 0)
    def _():
        acc_ref[...] = jnp.zeros_like(acc_ref)

    r_ds = bd // bs

    @pl.when((s >= d * r_ds) & (s < (d + 1) * r_ds))
    def _():
        ybuf_ref[:, pl.ds((s - d * r_ds) * bs, bs)] = y

    @pl.when(row_lim >= bs)
    def _():
        acc_ref[...] += jnp.dot(y, e_ref[...], preferred_element_type=jnp.float32)

    @pl.when(row_lim < bs)
    def _():
        erows = jax.lax.broadcasted_iota(jnp.int32, e_ref.shape, 0)
        e = jnp.where(erows < row_lim, e_ref[...], jnp.zeros((), e_ref.dtype))
        acc_ref[...] += jnp.dot(y, e, preferred_element_type=jnp.float32)

    @pl.when(s == ns - 1)
    def _():
        yself = ybuf_ref[...].astype(jnp.float32)
        r = dinv_d_ref[...] * (acc_ref[...] + yself) + b_ref[0]
        r = jnp.where(l < nl - 1, jnp.maximum(r, 0.0), r)

        lanes = jax.lax.broadcasted_iota(jnp.int32, r.shape, 1)
        r_masked = jnp.where(lanes < n - d * bd, r, 0.0)

        @pl.when(l == 0)
        def _():
            xa_ref[:, pl.ds(d * bd, bd)] = r_masked

        @pl.when(l == 1)
        def _():
            xb_ref[:, pl.ds(d * bd, bd)] = r_masked

        @pl.when(l == nl - 1)
        def _():
            out_ref[...] = r.T


def _gcn_layers(wstack, bstack, nf, e_c, dinv_p, bs, bd):
    nl, h = wstack.shape[0], wstack.shape[1]
    n = e_c.shape[0]
    np_ = dinv_p.shape[1]
    nd, ns = np_ // bd, np_ // bs
    return pl.pallas_call(
        functools.partial(
            _gcn_body, n=n, nl=nl, nd=nd, ns=ns, bs=bs, bd=bd
        ),
        grid=(nl, nd, ns),
        in_specs=[
            pl.BlockSpec((1, h, h), lambda l, d, s: (l, 0, 0)),
            pl.BlockSpec(
                (bs, h), lambda l, d, s: (jnp.where(l == 0, s, 0), 0)
            ),
            pl.BlockSpec((bs, bd), lambda l, d, s: (s, d)),
            pl.BlockSpec((1, bs), lambda l, d, s: (0, s)),
            pl.BlockSpec((1, bd), lambda l, d, s: (0, d)),
            pl.BlockSpec((1, h, 1), lambda l, d, s: (l, 0, 0)),
        ],
        out_specs=pl.BlockSpec((bd, h), lambda l, d, s: (d, 0)),
        out_shape=jax.ShapeDtypeStruct((n, h), jnp.float32),
        scratch_shapes=[
            pltpu.VMEM((h, np_), jnp.float32),
            pltpu.VMEM((h, np_), jnp.float32),
            pltpu.VMEM((h, bd), jnp.float32),
            pltpu.VMEM((h, bd), _ETYPE),
        ],
    )(wstack, nf, e_c, dinv_p, dinv_p, bstack)


def kernel(node_features, edges, W1, b1, W2, b2, W3, b3):
    n, dim = node_features.shape
    bs = min(1280, -(-n // 128) * 128)
    np_ = -(-n // bs) * bs
    bd = min(4 * bs, np_)

    csum_bs = 1
    for c in range(16, min(n, 400) + 1, 16):
        if n % c == 0:
            csum_bs = c
    dinv, e_c = _colsum_dinv(edges, bs=csum_bs)

    dinv_p = jnp.pad(dinv, ((0, 0), (0, np_ - n)))
    wstack = jnp.stack([W1.T, W2.T, W3.T])
    bstack = jnp.stack([b1, b2, b3]).reshape(3, -1, 1)
    return _gcn_layers(wstack, bstack, node_features, e_c, dinv_p, bs, bd)
